# Initial kernel scaffold; baseline (speedup 1.0000x reference)
import functools

import jax
import jax.numpy as jnp
from jax import lax
from jax.experimental import pallas as pl
from jax.experimental.pallas import tpu as pltpu
from jax.experimental.pallas import tpu_sc as plsc

NN = 10000
EE = 20000
NNZ = 320000
DD = 128
DH = 64
NC = 2
NS = 16
LANES = 16

PAIRS_PER_TILE = NNZ // NS
CHUNK = 128
NFULL = PAIRS_PER_TILE // CHUNK
TAIL = PAIRS_PER_TILE - NFULL * CHUNK

EBLK = 1280
SBLK = 160
LAST_EBLK = EE - 15 * EBLK

EZERO_FULL = (EE // NS) // CHUNK
EZERO_TAIL = (EE // NS) - EZERO_FULL * CHUNK
NZERO_FULL = (NN // NS) // CHUNK
NZERO_TAIL = (NN // NS) - NZERO_FULL * CHUNK


def _sc_body(xlo, xhi, vert, edg, dege, out,
             vidx, eidx, vidx_t, eidx_t, rows, rows_t,
             cnt_priv, cntm, tbuf, dbuf, sbuf, srows,
             acc_e, acc_v, stage, sem):
    cid = lax.axis_index("c")
    sid = lax.axis_index("s")
    zf16 = jnp.zeros((LANES,), jnp.float32)
    ones16 = jnp.ones((LANES,), jnp.float32)

    def _zrow(i, _):
        for j in range(DH // LANES):
            rows[i, pl.ds(j * LANES, LANES)] = zf16
        return 0
    lax.fori_loop(0, CHUNK, _zrow, 0)

    def _zcnt(i, _):
        cnt_priv[pl.ds(i * LANES, LANES)] = zf16
        return 0
    lax.fori_loop(0, EE // LANES, _zcnt, 0)

    def _zcntm(i, _):
        cntm[pl.ds(i * LANES, LANES)] = zf16
        return 0
    lax.fori_loop(0, EBLK // LANES, _zcntm, 0)

    e0 = sid * (EE // NS)
    def _zacc_e(k, _):
        pltpu.sync_copy(rows, acc_e.at[pl.ds(e0 + k * CHUNK, CHUNK)])
        return 0
    lax.fori_loop(0, EZERO_FULL, _zacc_e, 0)
    pltpu.sync_copy(rows.at[pl.ds(0, EZERO_TAIL)],
                    acc_e.at[pl.ds(e0 + EZERO_FULL * CHUNK, EZERO_TAIL)])

    n0 = sid * (NN // NS)
    def _zacc_v(k, _):
        pltpu.sync_copy(rows, acc_v.at[pl.ds(n0 + k * CHUNK, CHUNK)])
        return 0
    lax.fori_loop(0, NZERO_FULL, _zacc_v, 0)
    pltpu.sync_copy(rows.at[pl.ds(0, NZERO_TAIL)],
                    acc_v.at[pl.ds(n0 + NZERO_FULL * CHUNK, NZERO_TAIL)])

    plsc.subcore_barrier()

    pair0 = sid * PAIRS_PER_TILE

    def _phase_a(x_ref):
        def _chunk(c, _):
            base = pair0 + c * CHUNK
            pltpu.sync_copy(vert.at[pl.ds(base, CHUNK)], vidx)
            pltpu.sync_copy(edg.at[pl.ds(base, CHUNK)], eidx)
            pltpu.async_copy(x_ref.at[vidx], rows, sem).wait()
            for j in range(CHUNK // LANES):
                ev = eidx[pl.ds(j * LANES, LANES)]
                plsc.addupdate_scatter(cnt_priv, [ev], ones16)
            pltpu.sync_copy(rows, acc_e.at[eidx], add=True)
            return 0
        lax.fori_loop(0, NFULL, _chunk, 0)
        tb = pair0 + NFULL * CHUNK
        pltpu.sync_copy(vert.at[pl.ds(tb, TAIL)], vidx_t)
        pltpu.sync_copy(edg.at[pl.ds(tb, TAIL)], eidx_t)
        pltpu.async_copy(x_ref.at[vidx_t], rows_t, sem).wait()
        for j in range(TAIL // LANES):
            ev = eidx_t[pl.ds(j * LANES, LANES)]
            plsc.addupdate_scatter(cnt_priv, [ev], ones16)
        pltpu.sync_copy(rows_t, acc_e.at[eidx_t], add=True)

    @pl.when(cid == 0)
    def _():
        _phase_a(xlo)

    @pl.when(cid == 1)
    def _():
        _phase_a(xhi)

    plsc.subcore_barrier()

    my_e0 = sid * EBLK
    nblk = jnp.where(sid < NS - 1, EBLK // SBLK, LAST_EBLK // SBLK)
    for r in range(NS // 4):
        @pl.when((sid >= 4 * r) & (sid < 4 * r + 4))
        def _():
            pltpu.sync_copy(cnt_priv, stage.at[sid - 4 * r])
        plsc.subcore_barrier()
        for j in range(4):
            def _mblk(k, _):
                pltpu.sync_copy(
                    stage.at[j].at[pl.ds(my_e0 + k * SBLK, SBLK)], tbuf)
                for v in range(SBLK // LANES):
                    o = k * SBLK + v * LANES
                    cntm[pl.ds(o, LANES)] = (
                        cntm[pl.ds(o, LANES)]
                        + tbuf[pl.ds(v * LANES, LANES)])
                return 0
            lax.fori_loop(0, nblk, _mblk, 0)
        plsc.subcore_barrier()

    @pl.when(sid < NS - 1)
    def _():
        pltpu.sync_copy(dege.at[pl.ds(my_e0, EBLK)], dbuf)

    @pl.when(sid == NS - 1)
    def _():
        pltpu.sync_copy(dege.at[pl.ds((NS - 1) * EBLK, LAST_EBLK)],
                        dbuf.at[pl.ds(0, LAST_EBLK)])

    def _scale_vec(i, _):
        o = i * LANES
        sbuf[pl.ds(o, LANES)] = dbuf[pl.ds(o, LANES)] / jnp.maximum(
            cntm[pl.ds(o, LANES)], 1.0)
        return 0
    lax.fori_loop(0, nblk * (SBLK // LANES), _scale_vec, 0)

    def _scale_blk(k, _):
        pltpu.sync_copy(acc_e.at[pl.ds(my_e0 + k * SBLK, SBLK)], srows)
        def _row(r, _):
            s = sbuf[k * SBLK + r]
            for j in range(DH // LANES):
                srows[r, pl.ds(j * LANES, LANES)] = (
                    srows[r, pl.ds(j * LANES, LANES)] * s)
            return 0
        lax.fori_loop(0, SBLK, _row, 0)
        pltpu.sync_copy(srows, acc_e.at[pl.ds(my_e0 + k * SBLK, SBLK)])
        return 0
    lax.fori_loop(0, nblk, _scale_blk, 0)

    plsc.subcore_barrier()

    def _chunk_b(c, _):
        base = pair0 + c * CHUNK
        pltpu.sync_copy(edg.at[pl.ds(base, CHUNK)], eidx)
        pltpu.async_copy(acc_e.at[eidx], rows, sem).wait()
        pltpu.sync_copy(vert.at[pl.ds(base, CHUNK)], vidx)
        pltpu.sync_copy(rows, acc_v.at[vidx], add=True)
        return 0
    lax.fori_loop(0, NFULL, _chunk_b, 0)
    tb = pair0 + NFULL * CHUNK
    pltpu.sync_copy(edg.at[pl.ds(tb, TAIL)], eidx_t)
    pltpu.async_copy(acc_e.at[eidx_t], rows_t, sem).wait()
    pltpu.sync_copy(vert.at[pl.ds(tb, TAIL)], vidx_t)
    pltpu.sync_copy(rows_t, acc_v.at[vidx_t], add=True)

    plsc.subcore_barrier()

    ob = cid * NN + n0
    def _wout(k, _):
        pltpu.sync_copy(acc_v.at[pl.ds(n0 + k * CHUNK, CHUNK)], rows)
        pltpu.sync_copy(rows, out.at[pl.ds(ob + k * CHUNK, CHUNK)])
        return 0
    lax.fori_loop(0, NZERO_FULL, _wout, 0)
    pltpu.sync_copy(acc_v.at[pl.ds(n0 + NZERO_FULL * CHUNK, NZERO_TAIL)],
                    rows.at[pl.ds(0, NZERO_TAIL)])
    pltpu.sync_copy(rows.at[pl.ds(0, NZERO_TAIL)],
                    out.at[pl.ds(ob + NZERO_FULL * CHUNK, NZERO_TAIL)])


_sc_kernel = pl.kernel(
    _sc_body,
    out_type=jax.ShapeDtypeStruct((NC * NN, DH), jnp.float32),
    mesh=plsc.VectorSubcoreMesh(core_axis_name="c", subcore_axis_name="s"),
    scratch_types=[
        pltpu.VMEM((CHUNK,), jnp.int32),
        pltpu.VMEM((CHUNK,), jnp.int32),
        pltpu.VMEM((TAIL,), jnp.int32),
        pltpu.VMEM((TAIL,), jnp.int32),
        pltpu.VMEM((CHUNK, DH), jnp.float32),
        pltpu.VMEM((TAIL, DH), jnp.float32),
        pltpu.VMEM((EE,), jnp.float32),
        pltpu.VMEM((EBLK,), jnp.float32),
        pltpu.VMEM((SBLK,), jnp.float32),
        pltpu.VMEM((EBLK,), jnp.float32),
        pltpu.VMEM((EBLK,), jnp.float32),
        pltpu.VMEM((SBLK, DH), jnp.float32),
        pltpu.VMEM_SHARED((EE, DH), jnp.float32),
        pltpu.VMEM_SHARED((NN, DH), jnp.float32),
        pltpu.VMEM_SHARED((4, EE), jnp.float32),
        pltpu.SemaphoreType.DMA,
    ],
)


def _tc_body(ab, xvlo, xvhi, x0, degv, w, out):
    alpha = ab[0]
    beta = ab[1]
    xv = jnp.concatenate([xvlo[...], xvhi[...]], axis=1)
    xi = (1.0 - alpha) * (xv * degv[...]) + alpha * x0[...]
    mm = lax.dot_general(xi, w[...], (((1,), (1,)), ((), ())),
                         preferred_element_type=jnp.float32,
                         precision=lax.Precision.HIGHEST)
    out[...] = (1.0 - beta) * xi + beta * mm


BR = 500


def _tc_combine(ab, xvlo, xvhi, x0, degv, w):
    grid = NN // BR
    return pl.pallas_call(
        _tc_body,
        grid=(grid,),
        in_specs=[
            pl.BlockSpec(memory_space=pltpu.SMEM),
            pl.BlockSpec((BR, DH), lambda i: (i, 0)),
            pl.BlockSpec((BR, DH), lambda i: (i, 0)),
            pl.BlockSpec((BR, DD), lambda i: (i, 0)),
            pl.BlockSpec((BR, 1), lambda i: (i, 0)),
            pl.BlockSpec((DD, DD), lambda i: (0, 0)),
        ],
        out_specs=pl.BlockSpec((BR, DD), lambda i: (i, 0)),
        out_shape=jax.ShapeDtypeStruct((NN, DD), jnp.float32),
    )(ab, xvlo, xvhi, x0, degv, w)


def kernel(X, X0, W, degE, degV, alpha, beta, vertex, edges):
    xlo = X[:, :DH]
    xhi = X[:, DH:]
    vert = vertex.astype(jnp.int32)
    edg = edges.astype(jnp.int32)
    dege = degE.reshape(EE)
    xv2 = _sc_kernel(xlo, xhi, vert, edg, dege)
    ab = jnp.stack([alpha, beta]).astype(jnp.float32)
    return _tc_combine(ab, xv2[:NN], xv2[NN:], X0, degV, W)


# speedup vs baseline: 7.8899x; 7.8899x over previous
import jax
import jax.numpy as jnp
from jax import lax
from jax.experimental import pallas as pl
from jax.experimental.pallas import tpu as pltpu
from jax.experimental.pallas import tpu_sc as plsc

NN = 10000
EE = 20000
NNZ = 320000
DD = 128
DH = 64
NC = 2
NS = 16
LANES = 16

CHUNK = 128
KSUP = 2
SROWS = KSUP * CHUNK
NCHUNKS = NNZ // CHUNK
NSUP = NCHUNKS // (KSUP * NS)
TAILC = NCHUNKS - NSUP * KSUP * NS

EPAD = 20480
NPAD = 10240
EPB = EPAD // NS
NPB = NPAD // NS

EBLK = 1280
LAST_FULL = 6
LAST_TAIL = 32
LAST_EBLK = EE - (NS - 1) * EBLK

NOUT_FULL = 5
LAST_NOUT_FULL = 3
LAST_NOUT_TAIL = 16

_SC_PARAMS = pltpu.CompilerParams(needs_layout_passes=False,
                                  use_tc_tiling_on_sc=False)
_MESH = plsc.VectorSubcoreMesh(core_axis_name="c", subcore_axis_name="s")


def _zero_rows(rows, n):
    zf16 = jnp.zeros((LANES,), jnp.float32)
    def _zrow(i, _):
        for j in range(DH // LANES):
            rows[i, pl.ds(j * LANES, LANES)] = zf16
        return 0
    lax.fori_loop(0, n, _zrow, 0)


def _sc_a_body(xlo, xhi, vert2, edg2, dege, out,
               vblk0, eblk0, vblk1, eblk1, rows0, rows1, ones, cbuf, dbuf,
               sbuf, acc_e, cnt_sh, sem0, sem1, semc):
    cid = lax.axis_index("c")
    sid = lax.axis_index("s")
    zf16 = jnp.zeros((LANES,), jnp.float32)
    ones16 = jnp.ones((LANES,), jnp.float32)
    last = sid == NS - 1

    _zero_rows(rows0, CHUNK)
    for j in range(CHUNK // LANES):
        ones[pl.ds(j * LANES, LANES)] = ones16
    def _zc(i, _):
        cbuf[pl.ds(i * LANES, LANES)] = zf16
        return 0
    lax.fori_loop(0, EBLK // LANES, _zc, 0)

    def _zacc_e(k, _):
        pltpu.sync_copy(rows0.at[pl.ds(0, CHUNK)],
                        acc_e.at[pl.ds(sid * EPB + k * CHUNK, CHUNK)])
        return 0
    lax.fori_loop(0, EPB // CHUNK, _zacc_e, 0)
    pltpu.sync_copy(cbuf, cnt_sh.at[pl.ds(sid * EPB, EPB)])

    plsc.subcore_barrier()

    sup0 = sid * NSUP

    def _phase_a(x_ref):
        def _fire(vblk, rows, sem):
            for k in range(KSUP):
                pltpu.async_copy(x_ref.at[vblk.at[k]],
                                 rows.at[pl.ds(k * CHUNK, CHUNK)], sem)

        def _drain(vblk, rows, sem):
            for k in range(KSUP):
                pltpu.make_async_copy(x_ref.at[vblk.at[k]],
                                      rows.at[pl.ds(k * CHUNK, CHUNK)],
                                      sem).wait()

        def _commit(eblk, rows):
            for k in range(KSUP):
                pltpu.async_copy(ones, cnt_sh.at[eblk.at[k]], semc, add=True)
            for k in range(KSUP):
                pltpu.async_copy(rows.at[pl.ds(k * CHUNK, CHUNK)],
                                 acc_e.at[eblk.at[k]], semc, add=True)
            for k in range(KSUP):
                pltpu.make_async_copy(ones, cnt_sh.at[eblk.at[0]],
                                      semc).wait()
            for k in range(KSUP):
                pltpu.make_async_copy(rows.at[pl.ds(k * CHUNK, CHUNK)],
                                      acc_e.at[eblk.at[k]], semc).wait()

        pltpu.sync_copy(vert2.at[pl.ds(sup0 * KSUP, KSUP)], vblk0)
        pltpu.sync_copy(edg2.at[pl.ds(sup0 * KSUP, KSUP)], eblk0)
        _fire(vblk0, rows0, sem0)

        def _pair(i, _):
            s = sup0 + 2 * i
            pltpu.sync_copy(vert2.at[pl.ds((s + 1) * KSUP, KSUP)], vblk1)
            pltpu.sync_copy(edg2.at[pl.ds((s + 1) * KSUP, KSUP)], eblk1)
            _fire(vblk1, rows1, sem1)
            _drain(vblk0, rows0, sem0)
            _commit(eblk0, rows0)
            @pl.when(2 * i + 2 < NSUP)
            def _():
                pltpu.sync_copy(vert2.at[pl.ds((s + 2) * KSUP, KSUP)], vblk0)
                pltpu.sync_copy(edg2.at[pl.ds((s + 2) * KSUP, KSUP)], eblk0)
                _fire(vblk0, rows0, sem0)
            _drain(vblk1, rows1, sem1)
            _commit(eblk1, rows1)
            return 0
        lax.fori_loop(0, NSUP // 2, _pair, 0)

        @pl.when(sid >= NS - TAILC)
        def _():
            c = NS * NSUP * KSUP + (sid - (NS - TAILC))
            pltpu.sync_copy(vert2.at[pl.ds(c, 1)], vblk0.at[pl.ds(0, 1)])
            pltpu.sync_copy(edg2.at[pl.ds(c, 1)], eblk0.at[pl.ds(0, 1)])
            pltpu.async_copy(x_ref.at[vblk0.at[0]],
                             rows0.at[pl.ds(0, CHUNK)], sem0)
            pltpu.make_async_copy(x_ref.at[vblk0.at[0]],
                                  rows0.at[pl.ds(0, CHUNK)], sem0).wait()
            pltpu.sync_copy(ones, cnt_sh.at[eblk0.at[0]], add=True)
            pltpu.sync_copy(rows0.at[pl.ds(0, CHUNK)],
                            acc_e.at[eblk0.at[0]], add=True)

    @pl.when(cid == 0)
    def _():
        _phase_a(xlo)

    @pl.when(cid == 1)
    def _():
        _phase_a(xhi)

    plsc.subcore_barrier()

    my_e0 = sid * EBLK
    nblk = jnp.where(last, LAST_FULL, EBLK // CHUNK)

    @pl.when(jnp.logical_not(last))
    def _():
        pltpu.sync_copy(dege.at[pl.ds(my_e0, EBLK)], dbuf)
        pltpu.sync_copy(cnt_sh.at[pl.ds(my_e0, EBLK)], cbuf)

    @pl.when(last)
    def _():
        pltpu.sync_copy(dege.at[pl.ds((NS - 1) * EBLK, LAST_EBLK)],
                        dbuf.at[pl.ds(0, LAST_EBLK)])
        pltpu.sync_copy(cnt_sh.at[pl.ds((NS - 1) * EBLK, LAST_EBLK)],
                        cbuf.at[pl.ds(0, LAST_EBLK)])

    nvec = jnp.where(last, LAST_EBLK // LANES, EBLK // LANES)

    def _scale_vec(i, _):
        o = i * LANES
        sbuf[pl.ds(o, LANES)] = dbuf[pl.ds(o, LANES)] / jnp.maximum(
            cbuf[pl.ds(o, LANES)], 1.0)
        return 0
    lax.fori_loop(0, nvec, _scale_vec, 0)

    ob = cid * EPAD + my_e0

    def _scale_grp(g, blk_off):
        sv = sbuf[pl.ds(blk_off + g * LANES, LANES)]
        for j in range(LANES):
            s = sv[j]
            for c in range(DH // LANES):
                rows0[g * LANES + j, pl.ds(c * LANES, LANES)] = (
                    rows0[g * LANES + j, pl.ds(c * LANES, LANES)] * s)

    def _scale_blk(k, _):
        pltpu.sync_copy(acc_e.at[pl.ds(my_e0 + k * CHUNK, CHUNK)],
                        rows0.at[pl.ds(0, CHUNK)])
        def _g(g, _):
            _scale_grp(g, k * CHUNK)
            return 0
        lax.fori_loop(0, CHUNK // LANES, _g, 0)
        pltpu.sync_copy(rows0.at[pl.ds(0, CHUNK)],
                        out.at[pl.ds(ob + k * CHUNK, CHUNK)])
        return 0
    lax.fori_loop(0, nblk, _scale_blk, 0)

    @pl.when(last)
    def _():
        o = LAST_FULL * CHUNK
        pltpu.sync_copy(acc_e.at[pl.ds(my_e0 + o, LAST_TAIL)],
                        rows0.at[pl.ds(0, LAST_TAIL)])
        def _g(g, _):
            _scale_grp(g, o)
            return 0
        lax.fori_loop(0, LAST_TAIL // LANES, _g, 0)
        pltpu.sync_copy(rows0.at[pl.ds(0, LAST_TAIL)],
                        out.at[pl.ds(ob + o, LAST_TAIL)])


_sc_a = pl.kernel(
    _sc_a_body,
    out_type=jax.ShapeDtypeStruct((NC * EPAD, DH), jnp.float32),
    mesh=_MESH,
    compiler_params=_SC_PARAMS,
    scratch_types=[
        pltpu.VMEM((KSUP, CHUNK), jnp.int32),
        pltpu.VMEM((KSUP, CHUNK), jnp.int32),
        pltpu.VMEM((KSUP, CHUNK), jnp.int32),
        pltpu.VMEM((KSUP, CHUNK), jnp.int32),
        pltpu.VMEM((SROWS, DH), jnp.float32),
        pltpu.VMEM((SROWS, DH), jnp.float32),
        pltpu.VMEM((CHUNK,), jnp.float32),
        pltpu.VMEM((EBLK,), jnp.float32),
        pltpu.VMEM((EBLK,), jnp.float32),
        pltpu.VMEM((EBLK,), jnp.float32),
        pltpu.VMEM_SHARED((EPAD, DH), jnp.float32),
        pltpu.VMEM_SHARED((EPAD,), jnp.float32),
        pltpu.SemaphoreType.DMA,
        pltpu.SemaphoreType.DMA,
        pltpu.SemaphoreType.DMA,
    ],
)


def _sc_b_body(xe2, vert2, edg2, out,
               vblk0, eblk0, vblk1, eblk1, rows0, rows1, acc_v,
               sem0, sem1, sems):
    cid = lax.axis_index("c")
    sid = lax.axis_index("s")
    last = sid == NS - 1

    _zero_rows(rows0, CHUNK)

    def _zacc_v(k, _):
        pltpu.sync_copy(rows0.at[pl.ds(0, CHUNK)],
                        acc_v.at[pl.ds(sid * NPB + k * CHUNK, CHUNK)])
        return 0
    lax.fori_loop(0, NPB // CHUNK, _zacc_v, 0)

    plsc.subcore_barrier()

    sup0 = sid * NSUP
    off = cid * EPAD

    def _load_idx(s, vblk, eblk):
        pltpu.sync_copy(vert2.at[pl.ds(s * KSUP, KSUP)], vblk)
        pltpu.sync_copy(edg2.at[pl.ds(s * KSUP, KSUP)], eblk)
        for k in range(KSUP):
            for j in range(CHUNK // LANES):
                eblk[k, pl.ds(j * LANES, LANES)] = (
                    eblk[k, pl.ds(j * LANES, LANES)] + off)

    def _fire(eblk, rows, sem):
        for k in range(KSUP):
            pltpu.async_copy(xe2.at[eblk.at[k]],
                             rows.at[pl.ds(k * CHUNK, CHUNK)], sem)

    def _drain(eblk, rows, sem):
        for k in range(KSUP):
            pltpu.make_async_copy(xe2.at[eblk.at[k]],
                                  rows.at[pl.ds(k * CHUNK, CHUNK)],
                                  sem).wait()

    def _commit(vblk, rows):
        for k in range(KSUP):
            pltpu.async_copy(rows.at[pl.ds(k * CHUNK, CHUNK)],
                             acc_v.at[vblk.at[k]], sems, add=True)
        for k in range(KSUP):
            pltpu.make_async_copy(rows.at[pl.ds(k * CHUNK, CHUNK)],
                                  acc_v.at[vblk.at[k]], sems).wait()

    _load_idx(sup0, vblk0, eblk0)
    _fire(eblk0, rows0, sem0)

    def _pair(i, _):
        s = sup0 + 2 * i
        _load_idx(s + 1, vblk1, eblk1)
        _fire(eblk1, rows1, sem1)
        _drain(eblk0, rows0, sem0)
        _commit(vblk0, rows0)
        @pl.when(2 * i + 2 < NSUP)
        def _():
            _load_idx(s + 2, vblk0, eblk0)
            _fire(eblk0, rows0, sem0)
        _drain(eblk1, rows1, sem1)
        _commit(vblk1, rows1)
        return 0
    lax.fori_loop(0, NSUP // 2, _pair, 0)

    @pl.when(sid >= NS - TAILC)
    def _():
        c = NS * NSUP * KSUP + (sid - (NS - TAILC))
        pltpu.sync_copy(vert2.at[pl.ds(c, 1)], vblk0.at[pl.ds(0, 1)])
        pltpu.sync_copy(edg2.at[pl.ds(c, 1)], eblk0.at[pl.ds(0, 1)])
        for j in range(CHUNK // LANES):
            eblk0[0, pl.ds(j * LANES, LANES)] = (
                eblk0[0, pl.ds(j * LANES, LANES)] + off)
        pltpu.async_copy(xe2.at[eblk0.at[0]],
                         rows0.at[pl.ds(0, CHUNK)], sem0)
        pltpu.make_async_copy(xe2.at[eblk0.at[0]],
                              rows0.at[pl.ds(0, CHUNK)], sem0).wait()
        pltpu.sync_copy(rows0.at[pl.ds(0, CHUNK)],
                        acc_v.at[vblk0.at[0]], add=True)

    plsc.subcore_barrier()

    n0 = sid * NPB
    ob = cid * NPAD + n0
    nout = jnp.where(last, LAST_NOUT_FULL, NOUT_FULL)

    def _wout(k, _):
        pltpu.sync_copy(acc_v.at[pl.ds(n0 + k * CHUNK, CHUNK)],
                        rows0.at[pl.ds(0, CHUNK)])
        pltpu.sync_copy(rows0.at[pl.ds(0, CHUNK)],
                        out.at[pl.ds(ob + k * CHUNK, CHUNK)])
        return 0
    lax.fori_loop(0, nout, _wout, 0)

    @pl.when(last)
    def _():
        o = LAST_NOUT_FULL * CHUNK
        pltpu.sync_copy(acc_v.at[pl.ds(n0 + o, LAST_NOUT_TAIL)],
                        rows0.at[pl.ds(0, LAST_NOUT_TAIL)])
        pltpu.sync_copy(rows0.at[pl.ds(0, LAST_NOUT_TAIL)],
                        out.at[pl.ds(ob + o, LAST_NOUT_TAIL)])


_sc_b = pl.kernel(
    _sc_b_body,
    out_type=jax.ShapeDtypeStruct((NC * NPAD, DH), jnp.float32),
    mesh=_MESH,
    compiler_params=_SC_PARAMS,
    scratch_types=[
        pltpu.VMEM((KSUP, CHUNK), jnp.int32),
        pltpu.VMEM((KSUP, CHUNK), jnp.int32),
        pltpu.VMEM((KSUP, CHUNK), jnp.int32),
        pltpu.VMEM((KSUP, CHUNK), jnp.int32),
        pltpu.VMEM((SROWS, DH), jnp.float32),
        pltpu.VMEM((SROWS, DH), jnp.float32),
        pltpu.VMEM_SHARED((NPAD, DH), jnp.float32),
        pltpu.SemaphoreType.DMA,
        pltpu.SemaphoreType.DMA,
        pltpu.SemaphoreType.DMA,
    ],
)


def _tc_body(ab, xvlo, xvhi, x0, degv, w, out):
    alpha = ab[0]
    beta = ab[1]
    xv = jnp.concatenate([xvlo[...], xvhi[...]], axis=1)
    xi = (1.0 - alpha) * (xv * degv[...]) + alpha * x0[...]
    mm = lax.dot_general(xi, w[...], (((1,), (1,)), ((), ())),
                         preferred_element_type=jnp.float32,
                         precision=lax.Precision.HIGHEST)
    out[...] = (1.0 - beta) * xi + beta * mm


BR = 400


def _tc_combine(ab, xvlo, xvhi, x0, degv, w):
    grid = NN // BR
    return pl.pallas_call(
        _tc_body,
        grid=(grid,),
        in_specs=[
            pl.BlockSpec(memory_space=pltpu.SMEM),
            pl.BlockSpec((BR, DH), lambda i: (i, 0)),
            pl.BlockSpec((BR, DH), lambda i: (i, 0)),
            pl.BlockSpec((BR, DD), lambda i: (i, 0)),
            pl.BlockSpec((BR, 1), lambda i: (i, 0)),
            pl.BlockSpec((DD, DD), lambda i: (0, 0)),
        ],
        out_specs=pl.BlockSpec((BR, DD), lambda i: (i, 0)),
        out_shape=jax.ShapeDtypeStruct((NN, DD), jnp.float32),
    )(ab, xvlo, xvhi, x0, degv, w)


def kernel(X, X0, W, degE, degV, alpha, beta, vertex, edges):
    xlo = X[:, :DH]
    xhi = X[:, DH:]
    vert2 = vertex.astype(jnp.int32).reshape(NCHUNKS, CHUNK)
    edg2 = edges.astype(jnp.int32).reshape(NCHUNKS, CHUNK)
    dege = degE.reshape(EE)
    xe2 = _sc_a(xlo, xhi, vert2, edg2, dege)
    xv2 = _sc_b(xe2, vert2, edg2)
    ab = jnp.stack([alpha, beta]).astype(jnp.float32)
    return _tc_combine(ab, xv2[:NN], xv2[NPAD:NPAD + NN], X0, degV, W)


# speedup vs baseline: 8.9157x; 1.1300x over previous
import jax
import jax.numpy as jnp
from jax import lax
from jax.experimental import pallas as pl
from jax.experimental.pallas import tpu as pltpu
from jax.experimental.pallas import tpu_sc as plsc

NN = 10000
EE = 20000
NNZ = 320000
DD = 128
DH = 64
NC = 2
NS = 16
LANES = 16

CHUNK = 128
KSUP = 2
SROWS = KSUP * CHUNK
NCHUNKS = NNZ // CHUNK
NSUP = NCHUNKS // (KSUP * NS)
TAILC = NCHUNKS - NSUP * KSUP * NS
KSUPB = 4
SROWSB = KSUPB * CHUNK
NSUPB = NCHUNKS // (KSUPB * NS)
assert NCHUNKS - NSUPB * KSUPB * NS == TAILC

EPAD = 20480
NPAD = 10240
EPB = EPAD // NS
NPB = NPAD // NS

EBLK = 1280
LAST_FULL = 6
LAST_TAIL = 32
LAST_EBLK = EE - (NS - 1) * EBLK

NOUT_FULL = 5
LAST_NOUT_FULL = 3
LAST_NOUT_TAIL = 16

_SC_PARAMS = pltpu.CompilerParams(needs_layout_passes=False,
                                  use_tc_tiling_on_sc=False)
_MESH = plsc.VectorSubcoreMesh(core_axis_name="c", subcore_axis_name="s")


def _zero_rows(rows, n):
    zf16 = jnp.zeros((LANES,), jnp.float32)
    def _zrow(i, _):
        for j in range(DH // LANES):
            rows[i, pl.ds(j * LANES, LANES)] = zf16
        return 0
    lax.fori_loop(0, n, _zrow, 0)


def _sc_a_body(x2, vert2, edg2, dege, out,
               vblk0, eblk0, vblk1, eblk1, rows0, rows1, ones, cbuf, dbuf,
               sbuf, acc_e, cnt_sh, sem0, sem1, semc):
    cid = lax.axis_index("c")
    sid = lax.axis_index("s")
    zf16 = jnp.zeros((LANES,), jnp.float32)
    ones16 = jnp.ones((LANES,), jnp.float32)
    last = sid == NS - 1

    _zero_rows(rows0, CHUNK)
    for j in range(CHUNK // LANES):
        ones[pl.ds(j * LANES, LANES)] = ones16
    def _zc(i, _):
        cbuf[pl.ds(i * LANES, LANES)] = zf16
        return 0
    lax.fori_loop(0, EBLK // LANES, _zc, 0)

    def _zacc_e(k, _):
        pltpu.sync_copy(rows0.at[pl.ds(0, CHUNK)],
                        acc_e.at[pl.ds(sid * EPB + k * CHUNK, CHUNK)])
        return 0
    lax.fori_loop(0, EPB // CHUNK, _zacc_e, 0)
    pltpu.sync_copy(cbuf, cnt_sh.at[pl.ds(sid * EPB, EPB)])

    plsc.subcore_barrier()

    sup0 = sid * NSUP

    def _load_idx(s, vblk, eblk):
        pltpu.sync_copy(vert2.at[pl.ds(s * KSUP, KSUP)], vblk)
        pltpu.sync_copy(edg2.at[pl.ds(s * KSUP, KSUP)], eblk)
        for k in range(KSUP):
            for j in range(CHUNK // LANES):
                vblk[k, pl.ds(j * LANES, LANES)] = (
                    vblk[k, pl.ds(j * LANES, LANES)] * 2 + cid)

    def _fire(vblk, rows, sem):
        for k in range(KSUP):
            pltpu.async_copy(x2.at[vblk.at[k]],
                             rows.at[pl.ds(k * CHUNK, CHUNK)], sem)

    def _drain(vblk, rows, sem):
        for k in range(KSUP):
            pltpu.make_async_copy(x2.at[vblk.at[k]],
                                  rows.at[pl.ds(k * CHUNK, CHUNK)],
                                  sem).wait()

    def _commit(eblk, rows):
        for k in range(KSUP):
            pltpu.async_copy(ones, cnt_sh.at[eblk.at[k]], semc, add=True)
        for k in range(KSUP):
            pltpu.async_copy(rows.at[pl.ds(k * CHUNK, CHUNK)],
                             acc_e.at[eblk.at[k]], semc, add=True)
        for k in range(KSUP):
            pltpu.make_async_copy(ones, cnt_sh.at[eblk.at[0]],
                                  semc).wait()
        for k in range(KSUP):
            pltpu.make_async_copy(rows.at[pl.ds(k * CHUNK, CHUNK)],
                                  acc_e.at[eblk.at[k]], semc).wait()

    _load_idx(sup0, vblk0, eblk0)
    _fire(vblk0, rows0, sem0)

    def _pair(i, _):
        s = sup0 + 2 * i
        _load_idx(s + 1, vblk1, eblk1)
        _fire(vblk1, rows1, sem1)
        _drain(vblk0, rows0, sem0)
        _commit(eblk0, rows0)
        @pl.when(2 * i + 2 < NSUP)
        def _():
            _load_idx(s + 2, vblk0, eblk0)
            _fire(vblk0, rows0, sem0)
        _drain(vblk1, rows1, sem1)
        _commit(eblk1, rows1)
        return 0
    lax.fori_loop(0, NSUP // 2, _pair, 0)

    @pl.when(sid >= NS - TAILC)
    def _():
        c = NS * NSUP * KSUP + (sid - (NS - TAILC))
        pltpu.sync_copy(vert2.at[pl.ds(c, 1)], vblk0.at[pl.ds(0, 1)])
        pltpu.sync_copy(edg2.at[pl.ds(c, 1)], eblk0.at[pl.ds(0, 1)])
        for j in range(CHUNK // LANES):
            vblk0[0, pl.ds(j * LANES, LANES)] = (
                vblk0[0, pl.ds(j * LANES, LANES)] * 2 + cid)
        pltpu.async_copy(x2.at[vblk0.at[0]],
                         rows0.at[pl.ds(0, CHUNK)], sem0)
        pltpu.make_async_copy(x2.at[vblk0.at[0]],
                              rows0.at[pl.ds(0, CHUNK)], sem0).wait()
        pltpu.sync_copy(ones, cnt_sh.at[eblk0.at[0]], add=True)
        pltpu.sync_copy(rows0.at[pl.ds(0, CHUNK)],
                        acc_e.at[eblk0.at[0]], add=True)

    plsc.subcore_barrier()

    my_e0 = sid * EBLK
    nblk = jnp.where(last, LAST_FULL, EBLK // CHUNK)

    @pl.when(jnp.logical_not(last))
    def _():
        pltpu.sync_copy(dege.at[pl.ds(my_e0, EBLK)], dbuf)
        pltpu.sync_copy(cnt_sh.at[pl.ds(my_e0, EBLK)], cbuf)

    @pl.when(last)
    def _():
        pltpu.sync_copy(dege.at[pl.ds((NS - 1) * EBLK, LAST_EBLK)],
                        dbuf.at[pl.ds(0, LAST_EBLK)])
        pltpu.sync_copy(cnt_sh.at[pl.ds((NS - 1) * EBLK, LAST_EBLK)],
                        cbuf.at[pl.ds(0, LAST_EBLK)])

    nvec = jnp.where(last, LAST_EBLK // LANES, EBLK // LANES)

    def _scale_vec(i, _):
        o = i * LANES
        sbuf[pl.ds(o, LANES)] = dbuf[pl.ds(o, LANES)] / jnp.maximum(
            cbuf[pl.ds(o, LANES)], 1.0)
        return 0
    lax.fori_loop(0, nvec, _scale_vec, 0)

    ob = cid * EPAD + my_e0

    def _scale_grp(g, blk_off):
        sv = sbuf[pl.ds(blk_off + g * LANES, LANES)]
        for j in range(LANES):
            s = sv[j]
            for c in range(DH // LANES):
                rows0[g * LANES + j, pl.ds(c * LANES, LANES)] = (
                    rows0[g * LANES + j, pl.ds(c * LANES, LANES)] * s)

    def _scale_blk(k, _):
        pltpu.sync_copy(acc_e.at[pl.ds(my_e0 + k * CHUNK, CHUNK)],
                        rows0.at[pl.ds(0, CHUNK)])
        def _g(g, _):
            _scale_grp(g, k * CHUNK)
            return 0
        lax.fori_loop(0, CHUNK // LANES, _g, 0)
        pltpu.sync_copy(rows0.at[pl.ds(0, CHUNK)],
                        out.at[pl.ds(ob + k * CHUNK, CHUNK)])
        return 0
    lax.fori_loop(0, nblk, _scale_blk, 0)

    @pl.when(last)
    def _():
        o = LAST_FULL * CHUNK
        pltpu.sync_copy(acc_e.at[pl.ds(my_e0 + o, LAST_TAIL)],
                        rows0.at[pl.ds(0, LAST_TAIL)])
        def _g(g, _):
            _scale_grp(g, o)
            return 0
        lax.fori_loop(0, LAST_TAIL // LANES, _g, 0)
        pltpu.sync_copy(rows0.at[pl.ds(0, LAST_TAIL)],
                        out.at[pl.ds(ob + o, LAST_TAIL)])


_sc_a = pl.kernel(
    _sc_a_body,
    out_type=jax.ShapeDtypeStruct((NC * EPAD, DH), jnp.float32),
    mesh=_MESH,
    compiler_params=_SC_PARAMS,
    scratch_types=[
        pltpu.VMEM((KSUP, CHUNK), jnp.int32),
        pltpu.VMEM((KSUP, CHUNK), jnp.int32),
        pltpu.VMEM((KSUP, CHUNK), jnp.int32),
        pltpu.VMEM((KSUP, CHUNK), jnp.int32),
        pltpu.VMEM((SROWS, DH), jnp.float32),
        pltpu.VMEM((SROWS, DH), jnp.float32),
        pltpu.VMEM((CHUNK,), jnp.float32),
        pltpu.VMEM((EBLK,), jnp.float32),
        pltpu.VMEM((EBLK,), jnp.float32),
        pltpu.VMEM((EBLK,), jnp.float32),
        pltpu.VMEM_SHARED((EPAD, DH), jnp.float32),
        pltpu.VMEM_SHARED((EPAD,), jnp.float32),
        pltpu.SemaphoreType.DMA,
        pltpu.SemaphoreType.DMA,
        pltpu.SemaphoreType.DMA,
    ],
)


def _sc_b_body(xe2, vert2, edg2, out,
               vblk0, eblk0, vblk1, eblk1, rows0, rows1, acc_v,
               sem0, sem1, sems):
    cid = lax.axis_index("c")
    sid = lax.axis_index("s")
    last = sid == NS - 1

    _zero_rows(rows0, CHUNK)

    def _zacc_v(k, _):
        pltpu.sync_copy(rows0.at[pl.ds(0, CHUNK)],
                        acc_v.at[pl.ds(sid * NPB + k * CHUNK, CHUNK)])
        return 0
    lax.fori_loop(0, NPB // CHUNK, _zacc_v, 0)

    plsc.subcore_barrier()

    sup0 = sid * NSUPB
    off = cid * EPAD

    def _load_idx(s, vblk, eblk):
        pltpu.sync_copy(vert2.at[pl.ds(s * KSUPB, KSUPB)], vblk)
        pltpu.sync_copy(edg2.at[pl.ds(s * KSUPB, KSUPB)], eblk)
        for k in range(KSUPB):
            for j in range(CHUNK // LANES):
                eblk[k, pl.ds(j * LANES, LANES)] = (
                    eblk[k, pl.ds(j * LANES, LANES)] + off)

    def _fire(eblk, rows, sem):
        for k in range(KSUPB):
            pltpu.async_copy(xe2.at[eblk.at[k]],
                             rows.at[pl.ds(k * CHUNK, CHUNK)], sem)

    def _drain(eblk, rows, sem):
        for k in range(KSUPB):
            pltpu.make_async_copy(xe2.at[eblk.at[k]],
                                  rows.at[pl.ds(k * CHUNK, CHUNK)],
                                  sem).wait()

    def _commit(vblk, rows):
        for k in range(KSUPB):
            pltpu.async_copy(rows.at[pl.ds(k * CHUNK, CHUNK)],
                             acc_v.at[vblk.at[k]], sems, add=True)
        for k in range(KSUPB):
            pltpu.make_async_copy(rows.at[pl.ds(k * CHUNK, CHUNK)],
                                  acc_v.at[vblk.at[k]], sems).wait()

    _load_idx(sup0, vblk0, eblk0)
    _fire(eblk0, rows0, sem0)

    def _pair(i, _):
        s = sup0 + 2 * i
        _load_idx(s + 1, vblk1, eblk1)
        _fire(eblk1, rows1, sem1)
        _drain(eblk0, rows0, sem0)
        _commit(vblk0, rows0)
        @pl.when(2 * i + 2 < NSUPB)
        def _():
            _load_idx(s + 2, vblk0, eblk0)
            _fire(eblk0, rows0, sem0)
        _drain(eblk1, rows1, sem1)
        _commit(vblk1, rows1)
        return 0
    lax.fori_loop(0, NSUPB // 2, _pair, 0)

    if NSUPB % 2 == 1:
        _drain(eblk0, rows0, sem0)
        _commit(vblk0, rows0)

    @pl.when(sid >= NS - TAILC)
    def _():
        c = NS * NSUPB * KSUPB + (sid - (NS - TAILC))
        pltpu.sync_copy(vert2.at[pl.ds(c, 1)], vblk0.at[pl.ds(0, 1)])
        pltpu.sync_copy(edg2.at[pl.ds(c, 1)], eblk0.at[pl.ds(0, 1)])
        for j in range(CHUNK // LANES):
            eblk0[0, pl.ds(j * LANES, LANES)] = (
                eblk0[0, pl.ds(j * LANES, LANES)] + off)
        pltpu.async_copy(xe2.at[eblk0.at[0]],
                         rows0.at[pl.ds(0, CHUNK)], sem0)
        pltpu.make_async_copy(xe2.at[eblk0.at[0]],
                              rows0.at[pl.ds(0, CHUNK)], sem0).wait()
        pltpu.sync_copy(rows0.at[pl.ds(0, CHUNK)],
                        acc_v.at[vblk0.at[0]], add=True)

    plsc.subcore_barrier()

    n0 = sid * NPB
    ob = cid * NPAD + n0
    nout = jnp.where(last, LAST_NOUT_FULL, NOUT_FULL)

    def _wout(k, _):
        pltpu.sync_copy(acc_v.at[pl.ds(n0 + k * CHUNK, CHUNK)],
                        rows0.at[pl.ds(0, CHUNK)])
        pltpu.sync_copy(rows0.at[pl.ds(0, CHUNK)],
                        out.at[pl.ds(ob + k * CHUNK, CHUNK)])
        return 0
    lax.fori_loop(0, nout, _wout, 0)

    @pl.when(last)
    def _():
        o = LAST_NOUT_FULL * CHUNK
        pltpu.sync_copy(acc_v.at[pl.ds(n0 + o, LAST_NOUT_TAIL)],
                        rows0.at[pl.ds(0, LAST_NOUT_TAIL)])
        pltpu.sync_copy(rows0.at[pl.ds(0, LAST_NOUT_TAIL)],
                        out.at[pl.ds(ob + o, LAST_NOUT_TAIL)])


_sc_b = pl.kernel(
    _sc_b_body,
    out_type=jax.ShapeDtypeStruct((NC * NPAD, DH), jnp.float32),
    mesh=_MESH,
    compiler_params=_SC_PARAMS,
    scratch_types=[
        pltpu.VMEM((KSUPB, CHUNK), jnp.int32),
        pltpu.VMEM((KSUPB, CHUNK), jnp.int32),
        pltpu.VMEM((KSUPB, CHUNK), jnp.int32),
        pltpu.VMEM((KSUPB, CHUNK), jnp.int32),
        pltpu.VMEM((SROWSB, DH), jnp.float32),
        pltpu.VMEM((SROWSB, DH), jnp.float32),
        pltpu.VMEM_SHARED((NPAD, DH), jnp.float32),
        pltpu.SemaphoreType.DMA,
        pltpu.SemaphoreType.DMA,
        pltpu.SemaphoreType.DMA,
    ],
)


def _tc_body(ab, xvlo, xvhi, x0, degv, w, out):
    alpha = ab[0]
    beta = ab[1]
    xv = jnp.concatenate([xvlo[...], xvhi[...]], axis=1)
    xi = (1.0 - alpha) * (xv * degv[...]) + alpha * x0[...]
    mm = lax.dot_general(xi, w[...], (((1,), (1,)), ((), ())),
                         preferred_element_type=jnp.float32,
                         precision=lax.Precision.HIGHEST)
    out[...] = (1.0 - beta) * xi + beta * mm


BR = 512


def _tc_combine(ab, xv2, x0, degv, w):
    grid = pl.cdiv(NN, BR)
    return pl.pallas_call(
        _tc_body,
        grid=(grid,),
        in_specs=[
            pl.BlockSpec(memory_space=pltpu.SMEM),
            pl.BlockSpec((BR, DH), lambda i: (i, 0)),
            pl.BlockSpec((BR, DH), lambda i: (i + NPAD // BR, 0)),
            pl.BlockSpec((BR, DD), lambda i: (i, 0)),
            pl.BlockSpec((BR, 1), lambda i: (i, 0)),
            pl.BlockSpec((DD, DD), lambda i: (0, 0)),
        ],
        out_specs=pl.BlockSpec((BR, DD), lambda i: (i, 0)),
        out_shape=jax.ShapeDtypeStruct((NN, DD), jnp.float32),
    )(ab, xv2, xv2, x0, degv, w)


def kernel(X, X0, W, degE, degV, alpha, beta, vertex, edges):
    x2 = X.reshape(NC * NN, DH)
    vert2 = vertex.astype(jnp.int32).reshape(NCHUNKS, CHUNK)
    edg2 = edges.astype(jnp.int32).reshape(NCHUNKS, CHUNK)
    dege = degE.reshape(EE)
    xe2 = _sc_a(x2, vert2, edg2, dege)
    xv2 = _sc_b(xe2, vert2, edg2)
    ab = jnp.stack([alpha, beta]).astype(jnp.float32)
    return _tc_combine(ab, xv2, X0, degV, W)


# speedup vs baseline: 9.7896x; 1.0980x over previous
import jax
import jax.numpy as jnp
from jax import lax
from jax.experimental import pallas as pl
from jax.experimental.pallas import tpu as pltpu
from jax.experimental.pallas import tpu_sc as plsc

NN = 10000
EE = 20000
NNZ = 320000
DD = 128
DH = 64
NC = 2
NS = 16
LANES = 16

CHUNK = 128
KSUP = 2
SROWS = KSUP * CHUNK
NCHUNKS = NNZ // CHUNK
NSUP = NCHUNKS // (KSUP * NS)
TAILC = NCHUNKS - NSUP * KSUP * NS
SUPER_GRP = 6
assert NSUP % SUPER_GRP == 0

EPAD = 20480
NPAD = 10240
EPB = EPAD // NS
NPB = NPAD // NS

EBLK = 1280
LAST_FULL = 6
LAST_TAIL = 32
LAST_EBLK = EE - (NS - 1) * EBLK

NOUT_FULL = 5
LAST_NOUT_FULL = 3
LAST_NOUT_TAIL = 16

_SC_PARAMS = pltpu.CompilerParams(needs_layout_passes=False,
                                  use_tc_tiling_on_sc=False)
_MESH = plsc.VectorSubcoreMesh(core_axis_name="c", subcore_axis_name="s")


def _zero_rows(rows, n):
    zf16 = jnp.zeros((LANES,), jnp.float32)
    def _zrow(i, _):
        for j in range(DH // LANES):
            rows[i, pl.ds(j * LANES, LANES)] = zf16
        return 0
    lax.fori_loop(0, n, _zrow, 0)


def _pair_pipeline(vert2, edg2, sup0, vbufs, ebufs, rows01, semi, sems01,
                   transform, fire1, commit):
    def _idx_issue(s, vb, eb):
        pltpu.async_copy(vert2.at[pl.ds(s * KSUP, KSUP)], vb, semi)
        pltpu.async_copy(edg2.at[pl.ds(s * KSUP, KSUP)], eb, semi)

    def _idx_wait(s, vb, eb):
        pltpu.make_async_copy(vert2.at[pl.ds(s * KSUP, KSUP)], vb,
                              semi).wait()
        pltpu.make_async_copy(edg2.at[pl.ds(s * KSUP, KSUP)], eb,
                              semi).wait()
        transform(vb, eb)

    _idx_issue(sup0, vbufs[0], ebufs[0])
    _idx_wait(sup0, vbufs[0], ebufs[0])
    _idx_issue(sup0 + 1, vbufs[1], ebufs[1])
    _idx_wait(sup0 + 1, vbufs[1], ebufs[1])
    _idx_issue(sup0 + 2, vbufs[2], ebufs[2])
    fire1(vbufs[0], ebufs[0], rows01[0], sems01[0])

    def _grp(p, _):
        s = sup0 + SUPER_GRP * p
        for j in range(SUPER_GRP):
            t = SUPER_GRP * p + j
            vj, ej = vbufs[j % 3], ebufs[j % 3]
            vn, en = vbufs[(j + 1) % 3], ebufs[(j + 1) % 3]
            vf, ef = vbufs[(j + 2) % 3], ebufs[(j + 2) % 3]
            rj, rn = rows01[j % 2], rows01[(j + 1) % 2]
            sj, sn = sems01[j % 2], sems01[(j + 1) % 2]
            @pl.when(t + 1 < NSUP)
            def _():
                fire1(vn, en, rn, sn)
            fire1(vj, ej, rj, sj, drain=True)
            commit(vj, ej, rj)
            @pl.when(t + 2 < NSUP)
            def _():
                _idx_wait(s + j + 2, vf, ef)
            @pl.when(t + 3 < NSUP)
            def _():
                _idx_issue(s + j + 3, vj, ej)
        return 0
    lax.fori_loop(0, NSUP // SUPER_GRP, _grp, 0)


def _sc_a_body(x2, vert2, edg2, dege, out,
               vb0, eb0, vb1, eb1, vb2, eb2, rows0, rows1, ones, cbuf,
               dbuf, sbuf, acc_e, cnt_sh, sem0, sem1, semc, semi):
    cid = lax.axis_index("c")
    sid = lax.axis_index("s")
    zf16 = jnp.zeros((LANES,), jnp.float32)
    ones16 = jnp.ones((LANES,), jnp.float32)
    last = sid == NS - 1

    _zero_rows(rows0, CHUNK)
    for j in range(CHUNK // LANES):
        ones[pl.ds(j * LANES, LANES)] = ones16
    def _zc(i, _):
        cbuf[pl.ds(i * LANES, LANES)] = zf16
        return 0
    lax.fori_loop(0, EBLK // LANES, _zc, 0)

    def _zacc_e(k, _):
        pltpu.sync_copy(rows0.at[pl.ds(0, CHUNK)],
                        acc_e.at[pl.ds(sid * EPB + k * CHUNK, CHUNK)])
        return 0
    lax.fori_loop(0, EPB // CHUNK, _zacc_e, 0)
    pltpu.sync_copy(cbuf, cnt_sh.at[pl.ds(sid * EPB, EPB)])

    plsc.subcore_barrier()

    sup0 = sid * NSUP

    def _transform(vb, eb):
        for k in range(KSUP):
            for j in range(CHUNK // LANES):
                vb[k, pl.ds(j * LANES, LANES)] = (
                    vb[k, pl.ds(j * LANES, LANES)] * 2 + cid)

    def _fire1(vb, eb, rows, sem, drain=False):
        for k in range(KSUP):
            if drain:
                pltpu.make_async_copy(x2.at[vb.at[k]],
                                      rows.at[pl.ds(k * CHUNK, CHUNK)],
                                      sem).wait()
            else:
                pltpu.async_copy(x2.at[vb.at[k]],
                                 rows.at[pl.ds(k * CHUNK, CHUNK)], sem)

    def _commit(vb, eb, rows):
        for k in range(KSUP):
            pltpu.async_copy(ones, cnt_sh.at[eb.at[k]], semc, add=True)
        for k in range(KSUP):
            pltpu.async_copy(rows.at[pl.ds(k * CHUNK, CHUNK)],
                             acc_e.at[eb.at[k]], semc, add=True)
        for k in range(KSUP):
            pltpu.make_async_copy(ones, cnt_sh.at[eb.at[0]], semc).wait()
        for k in range(KSUP):
            pltpu.make_async_copy(rows.at[pl.ds(k * CHUNK, CHUNK)],
                                  acc_e.at[eb.at[k]], semc).wait()

    _pair_pipeline(vert2, edg2, sup0, (vb0, vb1, vb2), (eb0, eb1, eb2),
                   (rows0, rows1), semi, (sem0, sem1),
                   _transform, _fire1, _commit)

    @pl.when(sid >= NS - TAILC)
    def _():
        c = NS * NSUP * KSUP + (sid - (NS - TAILC))
        pltpu.sync_copy(vert2.at[pl.ds(c, 1)], vb0.at[pl.ds(0, 1)])
        pltpu.sync_copy(edg2.at[pl.ds(c, 1)], eb0.at[pl.ds(0, 1)])
        for j in range(CHUNK // LANES):
            vb0[0, pl.ds(j * LANES, LANES)] = (
                vb0[0, pl.ds(j * LANES, LANES)] * 2 + cid)
        pltpu.async_copy(x2.at[vb0.at[0]],
                         rows0.at[pl.ds(0, CHUNK)], sem0)
        pltpu.make_async_copy(x2.at[vb0.at[0]],
                              rows0.at[pl.ds(0, CHUNK)], sem0).wait()
        pltpu.sync_copy(ones, cnt_sh.at[eb0.at[0]], add=True)
        pltpu.sync_copy(rows0.at[pl.ds(0, CHUNK)],
                        acc_e.at[eb0.at[0]], add=True)

    plsc.subcore_barrier()

    my_e0 = sid * EBLK
    nblk = jnp.where(last, LAST_FULL, EBLK // CHUNK)

    @pl.when(jnp.logical_not(last))
    def _():
        pltpu.sync_copy(dege.at[pl.ds(my_e0, EBLK)], dbuf)
        pltpu.sync_copy(cnt_sh.at[pl.ds(my_e0, EBLK)], cbuf)

    @pl.when(last)
    def _():
        pltpu.sync_copy(dege.at[pl.ds((NS - 1) * EBLK, LAST_EBLK)],
                        dbuf.at[pl.ds(0, LAST_EBLK)])
        pltpu.sync_copy(cnt_sh.at[pl.ds((NS - 1) * EBLK, LAST_EBLK)],
                        cbuf.at[pl.ds(0, LAST_EBLK)])

    nvec = jnp.where(last, LAST_EBLK // LANES, EBLK // LANES)

    def _scale_vec(i, _):
        o = i * LANES
        sbuf[pl.ds(o, LANES)] = dbuf[pl.ds(o, LANES)] / jnp.maximum(
            cbuf[pl.ds(o, LANES)], 1.0)
        return 0
    lax.fori_loop(0, nvec, _scale_vec, 0)

    ob = cid * EPAD + my_e0

    def _scale_grp(g, blk_off):
        sv = sbuf[pl.ds(blk_off + g * LANES, LANES)]
        for j in range(LANES):
            s = sv[j]
            for c in range(DH // LANES):
                rows0[g * LANES + j, pl.ds(c * LANES, LANES)] = (
                    rows0[g * LANES + j, pl.ds(c * LANES, LANES)] * s)

    def _scale_blk(k, _):
        pltpu.sync_copy(acc_e.at[pl.ds(my_e0 + k * CHUNK, CHUNK)],
                        rows0.at[pl.ds(0, CHUNK)])
        def _g(g, _):
            _scale_grp(g, k * CHUNK)
            return 0
        lax.fori_loop(0, CHUNK // LANES, _g, 0)
        pltpu.sync_copy(rows0.at[pl.ds(0, CHUNK)],
                        out.at[pl.ds(ob + k * CHUNK, CHUNK)])
        return 0
    lax.fori_loop(0, nblk, _scale_blk, 0)

    @pl.when(last)
    def _():
        o = LAST_FULL * CHUNK
        pltpu.sync_copy(acc_e.at[pl.ds(my_e0 + o, LAST_TAIL)],
                        rows0.at[pl.ds(0, LAST_TAIL)])
        def _g(g, _):
            _scale_grp(g, o)
            return 0
        lax.fori_loop(0, LAST_TAIL // LANES, _g, 0)
        pltpu.sync_copy(rows0.at[pl.ds(0, LAST_TAIL)],
                        out.at[pl.ds(ob + o, LAST_TAIL)])


_sc_a = pl.kernel(
    _sc_a_body,
    out_type=jax.ShapeDtypeStruct((NC * EPAD, DH), jnp.float32),
    mesh=_MESH,
    compiler_params=_SC_PARAMS,
    scratch_types=[
        pltpu.VMEM((KSUP, CHUNK), jnp.int32),
        pltpu.VMEM((KSUP, CHUNK), jnp.int32),
        pltpu.VMEM((KSUP, CHUNK), jnp.int32),
        pltpu.VMEM((KSUP, CHUNK), jnp.int32),
        pltpu.VMEM((KSUP, CHUNK), jnp.int32),
        pltpu.VMEM((KSUP, CHUNK), jnp.int32),
        pltpu.VMEM((SROWS, DH), jnp.float32),
        pltpu.VMEM((SROWS, DH), jnp.float32),
        pltpu.VMEM((CHUNK,), jnp.float32),
        pltpu.VMEM((EBLK,), jnp.float32),
        pltpu.VMEM((EBLK,), jnp.float32),
        pltpu.VMEM((EBLK,), jnp.float32),
        pltpu.VMEM_SHARED((EPAD, DH), jnp.float32),
        pltpu.VMEM_SHARED((EPAD,), jnp.float32),
        pltpu.SemaphoreType.DMA,
        pltpu.SemaphoreType.DMA,
        pltpu.SemaphoreType.DMA,
        pltpu.SemaphoreType.DMA,
    ],
)


def _sc_b_body(xe2, vert2, edg2, out,
               vb0, eb0, vb1, eb1, vb2, eb2, rows0, rows1, acc_v,
               sem0, sem1, sems, semi):
    cid = lax.axis_index("c")
    sid = lax.axis_index("s")
    last = sid == NS - 1

    _zero_rows(rows0, CHUNK)

    def _zacc_v(k, _):
        pltpu.sync_copy(rows0.at[pl.ds(0, CHUNK)],
                        acc_v.at[pl.ds(sid * NPB + k * CHUNK, CHUNK)])
        return 0
    lax.fori_loop(0, NPB // CHUNK, _zacc_v, 0)

    plsc.subcore_barrier()

    sup0 = sid * NSUP
    off = cid * EPAD

    def _transform(vb, eb):
        for k in range(KSUP):
            for j in range(CHUNK // LANES):
                eb[k, pl.ds(j * LANES, LANES)] = (
                    eb[k, pl.ds(j * LANES, LANES)] + off)

    def _fire1(vb, eb, rows, sem, drain=False):
        for k in range(KSUP):
            if drain:
                pltpu.make_async_copy(xe2.at[eb.at[k]],
                                      rows.at[pl.ds(k * CHUNK, CHUNK)],
                                      sem).wait()
            else:
                pltpu.async_copy(xe2.at[eb.at[k]],
                                 rows.at[pl.ds(k * CHUNK, CHUNK)], sem)

    def _commit(vb, eb, rows):
        for k in range(KSUP):
            pltpu.async_copy(rows.at[pl.ds(k * CHUNK, CHUNK)],
                             acc_v.at[vb.at[k]], sems, add=True)
        for k in range(KSUP):
            pltpu.make_async_copy(rows.at[pl.ds(k * CHUNK, CHUNK)],
                                  acc_v.at[vb.at[k]], sems).wait()

    _pair_pipeline(vert2, edg2, sup0, (vb0, vb1, vb2), (eb0, eb1, eb2),
                   (rows0, rows1), semi, (sem0, sem1),
                   _transform, _fire1, _commit)

    @pl.when(sid >= NS - TAILC)
    def _():
        c = NS * NSUP * KSUP + (sid - (NS - TAILC))
        pltpu.sync_copy(vert2.at[pl.ds(c, 1)], vb0.at[pl.ds(0, 1)])
        pltpu.sync_copy(edg2.at[pl.ds(c, 1)], eb0.at[pl.ds(0, 1)])
        for j in range(CHUNK // LANES):
            eb0[0, pl.ds(j * LANES, LANES)] = (
                eb0[0, pl.ds(j * LANES, LANES)] + off)
        pltpu.async_copy(xe2.at[eb0.at[0]],
                         rows0.at[pl.ds(0, CHUNK)], sem0)
        pltpu.make_async_copy(xe2.at[eb0.at[0]],
                              rows0.at[pl.ds(0, CHUNK)], sem0).wait()
        pltpu.sync_copy(rows0.at[pl.ds(0, CHUNK)],
                        acc_v.at[vb0.at[0]], add=True)

    plsc.subcore_barrier()

    n0 = sid * NPB
    ob = cid * NPAD + n0
    nout = jnp.where(last, LAST_NOUT_FULL, NOUT_FULL)

    def _wout(k, _):
        pltpu.sync_copy(acc_v.at[pl.ds(n0 + k * CHUNK, CHUNK)],
                        rows0.at[pl.ds(0, CHUNK)])
        pltpu.sync_copy(rows0.at[pl.ds(0, CHUNK)],
                        out.at[pl.ds(ob + k * CHUNK, CHUNK)])
        return 0
    lax.fori_loop(0, nout, _wout, 0)

    @pl.when(last)
    def _():
        o = LAST_NOUT_FULL * CHUNK
        pltpu.sync_copy(acc_v.at[pl.ds(n0 + o, LAST_NOUT_TAIL)],
                        rows0.at[pl.ds(0, LAST_NOUT_TAIL)])
        pltpu.sync_copy(rows0.at[pl.ds(0, LAST_NOUT_TAIL)],
                        out.at[pl.ds(ob + o, LAST_NOUT_TAIL)])


_sc_b = pl.kernel(
    _sc_b_body,
    out_type=jax.ShapeDtypeStruct((NC * NPAD, DH), jnp.float32),
    mesh=_MESH,
    compiler_params=_SC_PARAMS,
    scratch_types=[
        pltpu.VMEM((KSUP, CHUNK), jnp.int32),
        pltpu.VMEM((KSUP, CHUNK), jnp.int32),
        pltpu.VMEM((KSUP, CHUNK), jnp.int32),
        pltpu.VMEM((KSUP, CHUNK), jnp.int32),
        pltpu.VMEM((KSUP, CHUNK), jnp.int32),
        pltpu.VMEM((KSUP, CHUNK), jnp.int32),
        pltpu.VMEM((SROWS, DH), jnp.float32),
        pltpu.VMEM((SROWS, DH), jnp.float32),
        pltpu.VMEM_SHARED((NPAD, DH), jnp.float32),
        pltpu.SemaphoreType.DMA,
        pltpu.SemaphoreType.DMA,
        pltpu.SemaphoreType.DMA,
        pltpu.SemaphoreType.DMA,
    ],
)


def _tc_body(ab, xvlo, xvhi, x0, degv, w, out):
    alpha = ab[0]
    beta = ab[1]
    xv = jnp.concatenate([xvlo[...], xvhi[...]], axis=1)
    xi = (1.0 - alpha) * (xv * degv[...]) + alpha * x0[...]
    mm = lax.dot_general(xi, w[...], (((1,), (1,)), ((), ())),
                         preferred_element_type=jnp.float32,
                         precision=lax.Precision.HIGHEST)
    out[...] = (1.0 - beta) * xi + beta * mm


BR = 512


def _tc_combine(ab, xv2, x0, degv, w):
    grid = pl.cdiv(NN, BR)
    return pl.pallas_call(
        _tc_body,
        grid=(grid,),
        in_specs=[
            pl.BlockSpec(memory_space=pltpu.SMEM),
            pl.BlockSpec((BR, DH), lambda i: (i, 0)),
            pl.BlockSpec((BR, DH), lambda i: (i + NPAD // BR, 0)),
            pl.BlockSpec((BR, DD), lambda i: (i, 0)),
            pl.BlockSpec((BR, 1), lambda i: (i, 0)),
            pl.BlockSpec((DD, DD), lambda i: (0, 0)),
        ],
        out_specs=pl.BlockSpec((BR, DD), lambda i: (i, 0)),
        out_shape=jax.ShapeDtypeStruct((NN, DD), jnp.float32),
    )(ab, xv2, xv2, x0, degv, w)


def kernel(X, X0, W, degE, degV, alpha, beta, vertex, edges):
    x2 = X.reshape(NC * NN, DH)
    vert2 = vertex.astype(jnp.int32).reshape(NCHUNKS, CHUNK)
    edg2 = edges.astype(jnp.int32).reshape(NCHUNKS, CHUNK)
    dege = degE.reshape(EE)
    xe2 = _sc_a(x2, vert2, edg2, dege)
    xv2 = _sc_b(xe2, vert2, edg2)
    ab = jnp.stack([alpha, beta]).astype(jnp.float32)
    return _tc_combine(ab, xv2, X0, degV, W)


# speedup vs baseline: 10.6603x; 1.0889x over previous
import jax
import jax.numpy as jnp
from jax import lax
from jax.experimental import pallas as pl
from jax.experimental.pallas import tpu as pltpu
from jax.experimental.pallas import tpu_sc as plsc

NN = 10000
EE = 20000
NNZ = 320000
DD = 128
DH = 64
NC = 2
NS = 16
LANES = 16

CHUNK = 128
KSUP = 2
SROWS = KSUP * CHUNK
NCHUNKS = NNZ // CHUNK
NSUP = NCHUNKS // (KSUP * NS)
TAILC = NCHUNKS - NSUP * KSUP * NS
SUPER_GRP = 6
assert NSUP % SUPER_GRP == 0

EPAD = 20480
NPAD = 10240
EPB = EPAD // NS
NPB = NPAD // NS

EBLK = 1280
LAST_FULL = 6
LAST_TAIL = 32
LAST_EBLK = EE - (NS - 1) * EBLK

NOUT_FULL = 5
LAST_NOUT_FULL = 3
LAST_NOUT_TAIL = 16

_SC_PARAMS = pltpu.CompilerParams(needs_layout_passes=False,
                                  use_tc_tiling_on_sc=False)
_MESH = plsc.VectorSubcoreMesh(core_axis_name="c", subcore_axis_name="s")


def _zero_rows(rows, n):
    zf16 = jnp.zeros((LANES,), jnp.float32)
    def _zrow(i, _):
        for j in range(DH // LANES):
            rows[i, pl.ds(j * LANES, LANES)] = zf16
        return 0
    lax.fori_loop(0, n, _zrow, 0)


def _pair_pipeline(vert2, edg2, sup0, vbufs, ebufs, rows01, semi, sems01,
                   transform, fire1, commit):
    def _idx_issue(s, vb, eb):
        pltpu.async_copy(vert2.at[pl.ds(s * KSUP, KSUP)], vb, semi)
        pltpu.async_copy(edg2.at[pl.ds(s * KSUP, KSUP)], eb, semi)

    def _idx_wait(s, vb, eb):
        pltpu.make_async_copy(vert2.at[pl.ds(s * KSUP, KSUP)], vb,
                              semi).wait()
        pltpu.make_async_copy(edg2.at[pl.ds(s * KSUP, KSUP)], eb,
                              semi).wait()
        transform(vb, eb)

    _idx_issue(sup0, vbufs[0], ebufs[0])
    _idx_wait(sup0, vbufs[0], ebufs[0])
    _idx_issue(sup0 + 1, vbufs[1], ebufs[1])
    _idx_wait(sup0 + 1, vbufs[1], ebufs[1])
    _idx_issue(sup0 + 2, vbufs[2], ebufs[2])
    fire1(vbufs[0], ebufs[0], rows01[0], sems01[0])

    def _grp(p, _):
        s = sup0 + SUPER_GRP * p
        for j in range(SUPER_GRP):
            t = SUPER_GRP * p + j
            vj, ej = vbufs[j % 3], ebufs[j % 3]
            vn, en = vbufs[(j + 1) % 3], ebufs[(j + 1) % 3]
            vf, ef = vbufs[(j + 2) % 3], ebufs[(j + 2) % 3]
            rj, rn = rows01[j % 2], rows01[(j + 1) % 2]
            sj, sn = sems01[j % 2], sems01[(j + 1) % 2]
            @pl.when(t + 1 < NSUP)
            def _():
                fire1(vn, en, rn, sn)
            fire1(vj, ej, rj, sj, drain=True)
            commit(vj, ej, rj)
            @pl.when(t + 2 < NSUP)
            def _():
                _idx_wait(s + j + 2, vf, ef)
            @pl.when(t + 3 < NSUP)
            def _():
                _idx_issue(s + j + 3, vj, ej)
        return 0
    lax.fori_loop(0, NSUP // SUPER_GRP, _grp, 0)


def _sc_a_body(x2, vert2, edg2, dege, out,
               vb0, eb0, vb1, eb1, vb2, eb2, rows0, rows1, ones, cbuf,
               dbuf, sbuf, acc_e, cnt_sh, sem0, sem1, semc, semi):
    cid = lax.axis_index("c")
    sid = lax.axis_index("s")
    zf16 = jnp.zeros((LANES,), jnp.float32)
    ones16 = jnp.ones((LANES,), jnp.float32)
    last = sid == NS - 1

    _zero_rows(rows0, CHUNK)
    for j in range(CHUNK // LANES):
        ones[pl.ds(j * LANES, LANES)] = ones16
    def _zc(i, _):
        cbuf[pl.ds(i * LANES, LANES)] = zf16
        return 0
    lax.fori_loop(0, EBLK // LANES, _zc, 0)

    def _zacc_e(k, _):
        pltpu.sync_copy(rows0.at[pl.ds(0, CHUNK)],
                        acc_e.at[pl.ds(sid * EPB + k * CHUNK, CHUNK)])
        return 0
    lax.fori_loop(0, EPB // CHUNK, _zacc_e, 0)
    pltpu.sync_copy(cbuf, cnt_sh.at[pl.ds(sid * EPB, EPB)])

    plsc.subcore_barrier()

    sup0 = sid * NSUP

    def _transform(vb, eb):
        for k in range(KSUP):
            for j in range(CHUNK // LANES):
                vb[k, pl.ds(j * LANES, LANES)] = (
                    vb[k, pl.ds(j * LANES, LANES)] * 2 + cid)

    def _fire1(vb, eb, rows, sem, drain=False):
        for k in range(KSUP):
            if drain:
                pltpu.make_async_copy(x2.at[vb.at[k]],
                                      rows.at[pl.ds(k * CHUNK, CHUNK)],
                                      sem).wait()
            else:
                pltpu.async_copy(x2.at[vb.at[k]],
                                 rows.at[pl.ds(k * CHUNK, CHUNK)], sem)

    def _commit(vb, eb, rows):
        for k in range(KSUP):
            pltpu.async_copy(ones, cnt_sh.at[eb.at[k]], semc, add=True)
        for k in range(KSUP):
            pltpu.async_copy(rows.at[pl.ds(k * CHUNK, CHUNK)],
                             acc_e.at[eb.at[k]], semc, add=True)
        for k in range(KSUP):
            pltpu.make_async_copy(ones, cnt_sh.at[eb.at[0]], semc).wait()
        for k in range(KSUP):
            pltpu.make_async_copy(rows.at[pl.ds(k * CHUNK, CHUNK)],
                                  acc_e.at[eb.at[k]], semc).wait()

    _pair_pipeline(vert2, edg2, sup0, (vb0, vb1, vb2), (eb0, eb1, eb2),
                   (rows0, rows1), semi, (sem0, sem1),
                   _transform, _fire1, _commit)

    @pl.when(sid >= NS - TAILC)
    def _():
        c = NS * NSUP * KSUP + (sid - (NS - TAILC))
        pltpu.sync_copy(vert2.at[pl.ds(c, 1)], vb0.at[pl.ds(0, 1)])
        pltpu.sync_copy(edg2.at[pl.ds(c, 1)], eb0.at[pl.ds(0, 1)])
        for j in range(CHUNK // LANES):
            vb0[0, pl.ds(j * LANES, LANES)] = (
                vb0[0, pl.ds(j * LANES, LANES)] * 2 + cid)
        pltpu.async_copy(x2.at[vb0.at[0]],
                         rows0.at[pl.ds(0, CHUNK)], sem0)
        pltpu.make_async_copy(x2.at[vb0.at[0]],
                              rows0.at[pl.ds(0, CHUNK)], sem0).wait()
        pltpu.sync_copy(ones, cnt_sh.at[eb0.at[0]], add=True)
        pltpu.sync_copy(rows0.at[pl.ds(0, CHUNK)],
                        acc_e.at[eb0.at[0]], add=True)

    plsc.subcore_barrier()

    my_e0 = sid * EBLK
    nblk = jnp.where(last, LAST_FULL, EBLK // CHUNK)

    @pl.when(jnp.logical_not(last))
    def _():
        pltpu.sync_copy(dege.at[pl.ds(my_e0, EBLK)], dbuf)
        pltpu.sync_copy(cnt_sh.at[pl.ds(my_e0, EBLK)], cbuf)

    @pl.when(last)
    def _():
        pltpu.sync_copy(dege.at[pl.ds((NS - 1) * EBLK, LAST_EBLK)],
                        dbuf.at[pl.ds(0, LAST_EBLK)])
        pltpu.sync_copy(cnt_sh.at[pl.ds((NS - 1) * EBLK, LAST_EBLK)],
                        cbuf.at[pl.ds(0, LAST_EBLK)])

    nvec = jnp.where(last, LAST_EBLK // LANES, EBLK // LANES)

    def _scale_vec(i, _):
        o = i * LANES
        sbuf[pl.ds(o, LANES)] = dbuf[pl.ds(o, LANES)] / jnp.maximum(
            cbuf[pl.ds(o, LANES)], 1.0)
        return 0
    lax.fori_loop(0, nvec, _scale_vec, 0)

    ob = cid * EPAD + my_e0

    def _scale_grp(g, blk_off):
        sv = sbuf[pl.ds(blk_off + g * LANES, LANES)]
        for j in range(LANES):
            s = sv[j]
            for c in range(DH // LANES):
                rows0[g * LANES + j, pl.ds(c * LANES, LANES)] = (
                    rows0[g * LANES + j, pl.ds(c * LANES, LANES)] * s)

    def _scale_blk(k, _):
        pltpu.sync_copy(acc_e.at[pl.ds(my_e0 + k * CHUNK, CHUNK)],
                        rows0.at[pl.ds(0, CHUNK)])
        def _g(g, _):
            _scale_grp(g, k * CHUNK)
            return 0
        lax.fori_loop(0, CHUNK // LANES, _g, 0)
        pltpu.sync_copy(rows0.at[pl.ds(0, CHUNK)],
                        out.at[pl.ds(ob + k * CHUNK, CHUNK)])
        return 0
    lax.fori_loop(0, nblk, _scale_blk, 0)

    @pl.when(last)
    def _():
        o = LAST_FULL * CHUNK
        pltpu.sync_copy(acc_e.at[pl.ds(my_e0 + o, LAST_TAIL)],
                        rows0.at[pl.ds(0, LAST_TAIL)])
        def _g(g, _):
            _scale_grp(g, o)
            return 0
        lax.fori_loop(0, LAST_TAIL // LANES, _g, 0)
        pltpu.sync_copy(rows0.at[pl.ds(0, LAST_TAIL)],
                        out.at[pl.ds(ob + o, LAST_TAIL)])


_sc_a = pl.kernel(
    _sc_a_body,
    out_type=jax.ShapeDtypeStruct((NC * EPAD, DH), jnp.float32),
    mesh=_MESH,
    compiler_params=_SC_PARAMS,
    scratch_types=[
        pltpu.VMEM((KSUP, CHUNK), jnp.int32),
        pltpu.VMEM((KSUP, CHUNK), jnp.int32),
        pltpu.VMEM((KSUP, CHUNK), jnp.int32),
        pltpu.VMEM((KSUP, CHUNK), jnp.int32),
        pltpu.VMEM((KSUP, CHUNK), jnp.int32),
        pltpu.VMEM((KSUP, CHUNK), jnp.int32),
        pltpu.VMEM((SROWS, DH), jnp.float32),
        pltpu.VMEM((SROWS, DH), jnp.float32),
        pltpu.VMEM((CHUNK,), jnp.float32),
        pltpu.VMEM((EBLK,), jnp.float32),
        pltpu.VMEM((EBLK,), jnp.float32),
        pltpu.VMEM((EBLK,), jnp.float32),
        pltpu.VMEM_SHARED((EPAD, DH), jnp.float32),
        pltpu.VMEM_SHARED((EPAD,), jnp.float32),
        pltpu.SemaphoreType.DMA,
        pltpu.SemaphoreType.DMA,
        pltpu.SemaphoreType.DMA,
        pltpu.SemaphoreType.DMA,
    ],
)


def _sc_b_body(xe2, vert2, edg2, out,
               vb0, eb0, vb1, eb1, vb2, eb2, vb3, eb3, vb4, eb4, vb5, eb5,
               rows0, rows1, rows2, acc_v, sem0, sem1, sem2, sems,
               semi0, semi1):
    cid = lax.axis_index("c")
    sid = lax.axis_index("s")
    last = sid == NS - 1

    _zero_rows(rows0, CHUNK)

    def _zacc_v(k, _):
        pltpu.sync_copy(rows0.at[pl.ds(0, CHUNK)],
                        acc_v.at[pl.ds(sid * NPB + k * CHUNK, CHUNK)])
        return 0
    lax.fori_loop(0, NPB // CHUNK, _zacc_v, 0)

    plsc.subcore_barrier()

    sup0 = sid * NSUP
    off = cid * EPAD
    vbufs = (vb0, vb1, vb2, vb3, vb4, vb5)
    ebufs = (eb0, eb1, eb2, eb3, eb4, eb5)
    rowsb = (rows0, rows1, rows2)
    semsb = (sem0, sem1, sem2)

    def _transform(vb, eb):
        for k in range(KSUP):
            for j in range(CHUNK // LANES):
                eb[k, pl.ds(j * LANES, LANES)] = (
                    eb[k, pl.ds(j * LANES, LANES)] + off)

    def _fire1(vb, eb, rows, sem, drain=False):
        for k in range(KSUP):
            if drain:
                pltpu.make_async_copy(xe2.at[eb.at[k]],
                                      rows.at[pl.ds(k * CHUNK, CHUNK)],
                                      sem).wait()
            else:
                pltpu.async_copy(xe2.at[eb.at[k]],
                                 rows.at[pl.ds(k * CHUNK, CHUNK)], sem)

    def _commit(vb, eb, rows):
        for k in range(KSUP):
            pltpu.async_copy(rows.at[pl.ds(k * CHUNK, CHUNK)],
                             acc_v.at[vb.at[k]], sems, add=True)
        for k in range(KSUP):
            pltpu.make_async_copy(rows.at[pl.ds(k * CHUNK, CHUNK)],
                                  acc_v.at[vb.at[k]], sems).wait()

    semib = (semi0, semi1)

    def _idx_issue(s, vb, eb, semi):
        pltpu.async_copy(vert2.at[pl.ds(s * KSUP, KSUP)], vb, semi)
        pltpu.async_copy(edg2.at[pl.ds(s * KSUP, KSUP)], eb, semi)

    def _idx_wait(s, vb, eb, semi):
        pltpu.make_async_copy(vert2.at[pl.ds(s * KSUP, KSUP)], vb,
                              semi).wait()
        pltpu.make_async_copy(edg2.at[pl.ds(s * KSUP, KSUP)], eb,
                              semi).wait()
        _transform(vb, eb)

    _idx_issue(sup0, vbufs[0], ebufs[0], semib[0])
    _idx_issue(sup0 + 1, vbufs[1], ebufs[1], semib[1])
    _idx_wait(sup0, vbufs[0], ebufs[0], semib[0])
    _idx_wait(sup0 + 1, vbufs[1], ebufs[1], semib[1])
    _idx_issue(sup0 + 2, vbufs[2], ebufs[2], semib[0])
    _idx_issue(sup0 + 3, vbufs[3], ebufs[3], semib[1])
    _fire1(vbufs[0], ebufs[0], rowsb[0], semsb[0])
    _fire1(vbufs[1], ebufs[1], rowsb[1], semsb[1])

    def _grp(p, _):
        s = sup0 + SUPER_GRP * p
        for j in range(SUPER_GRP):
            t = SUPER_GRP * p + j
            @pl.when(t + 2 < NSUP)
            def _():
                _idx_wait(s + j + 2, vbufs[(j + 2) % 6], ebufs[(j + 2) % 6],
                          semib[j % 2])
                _fire1(vbufs[(j + 2) % 6], ebufs[(j + 2) % 6],
                       rowsb[(j + 2) % 3], semsb[(j + 2) % 3])
            @pl.when(t + 4 < NSUP)
            def _():
                _idx_issue(s + j + 4, vbufs[(j + 4) % 6], ebufs[(j + 4) % 6],
                           semib[j % 2])
            _fire1(vbufs[j % 6], ebufs[j % 6], rowsb[j % 3], semsb[j % 3],
                   drain=True)
            _commit(vbufs[j % 6], ebufs[j % 6], rowsb[j % 3])
        return 0
    lax.fori_loop(0, NSUP // SUPER_GRP, _grp, 0)

    @pl.when(sid >= NS - TAILC)
    def _():
        c = NS * NSUP * KSUP + (sid - (NS - TAILC))
        pltpu.sync_copy(vert2.at[pl.ds(c, 1)], vb0.at[pl.ds(0, 1)])
        pltpu.sync_copy(edg2.at[pl.ds(c, 1)], eb0.at[pl.ds(0, 1)])
        for j in range(CHUNK // LANES):
            eb0[0, pl.ds(j * LANES, LANES)] = (
                eb0[0, pl.ds(j * LANES, LANES)] + off)
        pltpu.async_copy(xe2.at[eb0.at[0]],
                         rows0.at[pl.ds(0, CHUNK)], sem0)
        pltpu.make_async_copy(xe2.at[eb0.at[0]],
                              rows0.at[pl.ds(0, CHUNK)], sem0).wait()
        pltpu.sync_copy(rows0.at[pl.ds(0, CHUNK)],
                        acc_v.at[vb0.at[0]], add=True)

    plsc.subcore_barrier()

    n0 = sid * NPB
    ob = cid * NPAD + n0
    nout = jnp.where(last, LAST_NOUT_FULL, NOUT_FULL)

    def _wout(k, _):
        pltpu.sync_copy(acc_v.at[pl.ds(n0 + k * CHUNK, CHUNK)],
                        rows0.at[pl.ds(0, CHUNK)])
        pltpu.sync_copy(rows0.at[pl.ds(0, CHUNK)],
                        out.at[pl.ds(ob + k * CHUNK, CHUNK)])
        return 0
    lax.fori_loop(0, nout, _wout, 0)

    @pl.when(last)
    def _():
        o = LAST_NOUT_FULL * CHUNK
        pltpu.sync_copy(acc_v.at[pl.ds(n0 + o, LAST_NOUT_TAIL)],
                        rows0.at[pl.ds(0, LAST_NOUT_TAIL)])
        pltpu.sync_copy(rows0.at[pl.ds(0, LAST_NOUT_TAIL)],
                        out.at[pl.ds(ob + o, LAST_NOUT_TAIL)])


_sc_b = pl.kernel(
    _sc_b_body,
    out_type=jax.ShapeDtypeStruct((NC * NPAD, DH), jnp.float32),
    mesh=_MESH,
    compiler_params=_SC_PARAMS,
    scratch_types=(
        [pltpu.VMEM((KSUP, CHUNK), jnp.int32)] * 12
        + [pltpu.VMEM((SROWS, DH), jnp.float32)] * 3
        + [pltpu.VMEM_SHARED((NPAD, DH), jnp.float32)]
        + [pltpu.SemaphoreType.DMA] * 6
    ),
)


def _tc_body(ab, xvlo, xvhi, x0, degv, w, out):
    alpha = ab[0]
    beta = ab[1]
    xv = jnp.concatenate([xvlo[...], xvhi[...]], axis=1)
    xi = (1.0 - alpha) * (xv * degv[...]) + alpha * x0[...]
    mm = lax.dot_general(xi, w[...], (((1,), (1,)), ((), ())),
                         preferred_element_type=jnp.float32,
                         precision=lax.Precision.HIGHEST)
    out[...] = (1.0 - beta) * xi + beta * mm


BR = 512


def _tc_combine(ab, xv2, x0, degv, w):
    grid = pl.cdiv(NN, BR)
    return pl.pallas_call(
        _tc_body,
        grid=(grid,),
        in_specs=[
            pl.BlockSpec(memory_space=pltpu.SMEM),
            pl.BlockSpec((BR, DH), lambda i: (i, 0)),
            pl.BlockSpec((BR, DH), lambda i: (i + NPAD // BR, 0)),
            pl.BlockSpec((BR, DD), lambda i: (i, 0)),
            pl.BlockSpec((BR, 1), lambda i: (i, 0)),
            pl.BlockSpec((DD, DD), lambda i: (0, 0)),
        ],
        out_specs=pl.BlockSpec((BR, DD), lambda i: (i, 0)),
        out_shape=jax.ShapeDtypeStruct((NN, DD), jnp.float32),
    )(ab, xv2, xv2, x0, degv, w)


def kernel(X, X0, W, degE, degV, alpha, beta, vertex, edges):
    x2 = X.reshape(NC * NN, DH)
    vert2 = vertex.astype(jnp.int32).reshape(NCHUNKS, CHUNK)
    edg2 = edges.astype(jnp.int32).reshape(NCHUNKS, CHUNK)
    dege = degE.reshape(EE)
    xe2 = _sc_a(x2, vert2, edg2, dege)
    xv2 = _sc_b(xe2, vert2, edg2)
    ab = jnp.stack([alpha, beta]).astype(jnp.float32)
    return _tc_combine(ab, xv2, X0, degV, W)
